# baseline (speedup 1.0000x reference)
import functools

import jax
import jax.numpy as jnp
from jax import lax
from jax.experimental import pallas as pl
from jax.experimental.pallas import tpu as pltpu
from jax.experimental.pallas import tpu_sc as plsc

N = 10000
E = 320000
IN = 128
OUT = 128
H = 8
C = OUT // H
NE = E + N
NC, NS, L = 2, 16, 16
NW = NC * NS
G = 128
KT = 84
EP = NW * G * KT
ER = EP // G
NEG = 0.2

_MESH = dict(core_axis_name="c", subcore_axis_name="s", num_cores=NC,
             num_subcores=NS)
_SC_PARAMS = pltpu.CompilerParams(needs_layout_passes=False,
                                  use_tc_tiling_on_sc=False)


def _mm_body(x_ref, w_ref, a_ref, xh_ref, ad_ref):
    xh = jnp.dot(x_ref[...], w_ref[...], preferred_element_type=jnp.float32)
    xh_ref[...] = xh
    ad_ref[...] = jnp.dot(xh, a_ref[...], preferred_element_type=jnp.float32)


def _stage_a(x, W, A):
    blk = 400
    return pl.pallas_call(
        _mm_body,
        grid=(N // blk,),
        in_specs=[
            pl.BlockSpec((blk, IN), lambda i: (i, 0)),
            pl.BlockSpec((IN, OUT), lambda i: (0, 0)),
            pl.BlockSpec((OUT, 2 * H), lambda i: (0, 0)),
        ],
        out_specs=[
            pl.BlockSpec((blk, OUT), lambda i: (i, 0)),
            pl.BlockSpec((blk, 2 * H), lambda i: (i, 0)),
        ],
        out_shape=[
            jax.ShapeDtypeStruct((N, OUT), jnp.float32),
            jax.ShapeDtypeStruct((N, 2 * H), jnp.float32),
        ],
    )(x, W, A)


@functools.partial(
    pl.kernel,
    out_type=(jax.ShapeDtypeStruct((EP, H), jnp.float32),
              jax.ShapeDtypeStruct((NC, N, H), jnp.float32)),
    mesh=plsc.VectorSubcoreMesh(**_MESH),
    compiler_params=_SC_PARAMS,
    scratch_types=[
        pltpu.VMEM((G,), jnp.int32), pltpu.VMEM((G,), jnp.int32),
        pltpu.VMEM((G,), jnp.int32), pltpu.VMEM((G,), jnp.int32),
        pltpu.VMEM((G, 2 * H), jnp.float32), pltpu.VMEM((G, 2 * H), jnp.float32),
        pltpu.VMEM((G, 2 * H), jnp.float32), pltpu.VMEM((G, 2 * H), jnp.float32),
        pltpu.VMEM((G, H), jnp.float32), pltpu.VMEM((G, H), jnp.float32),
        pltpu.VMEM_SHARED((N, H), jnp.float32),
        pltpu.SemaphoreType.DMA, pltpu.SemaphoreType.DMA,
        pltpu.SemaphoreType.DMA, pltpu.SemaphoreType.DMA,
        pltpu.SemaphoreType.DMA, pltpu.SemaphoreType.DMA,
    ],
)
def _pass1(ad_hbm, srcp_hbm, dstp_hbm, zero8_hbm, ex_hbm, dpart_hbm,
           srcA, srcB, dstA, dstB, adsA, adsB, addA, addB, exA, exB, den_sp,
           semIA, semIB, semGA, semGB, semSA, semSB):
    c = lax.axis_index("c")
    s = lax.axis_index("s")
    wid = c * NS + s

    @pl.when(s == 0)
    def _init():
        pltpu.sync_copy(zero8_hbm, den_sp)

    plsc.subcore_barrier()
    iota = lax.iota(jnp.int32, L)

    SL = (dict(src=srcA, dst=dstA, ads=adsA, add=addA, ex=exA,
               semI=semIA, semG=semGA, semS=semSA),
          dict(src=srcB, dst=dstB, ads=adsB, add=addB, ex=exB,
               semI=semIB, semG=semGB, semS=semSB))

    def issue_idx(k, S):
        kr = wid * KT + k
        pltpu.async_copy(srcp_hbm.at[kr], S["src"], S["semI"])
        pltpu.async_copy(dstp_hbm.at[kr], S["dst"], S["semI"])

    def drain_idx(S):
        pltpu.make_async_copy(srcp_hbm.at[0], S["src"], S["semI"]).wait()
        pltpu.make_async_copy(dstp_hbm.at[0], S["dst"], S["semI"]).wait()

    def issue_gather(S):
        pltpu.async_copy(ad_hbm.at[S["src"]], S["ads"], S["semG"])
        pltpu.async_copy(ad_hbm.at[S["dst"]], S["add"], S["semG"])

    def drain_gather(S):
        pltpu.make_async_copy(ad_hbm.at[pl.ds(0, G)], S["ads"], S["semG"]).wait()
        pltpu.make_async_copy(ad_hbm.at[pl.ds(0, G)], S["add"], S["semG"]).wait()

    def compute(k, S):
        base_e = (wid * KT + k) * G
        ads_v, add_v, ex_v = S["ads"], S["add"], S["ex"]

        def group(g, carry2):
            e_loc = g * L + iota
            mask = (base_e + e_loc) < NE
            for h in range(H):
                hs = jnp.full((L,), h, jnp.int32)
                hd = jnp.full((L,), H + h, jnp.int32)
                a = (plsc.load_gather(ads_v, [e_loc, hs])
                     + plsc.load_gather(add_v, [e_loc, hd]))
                a = jnp.where(a < 0, a * NEG, a)
                exv = jnp.where(mask, jnp.exp(a), 0.0)
                plsc.store_scatter(ex_v, [e_loc, hs], exv)
            return carry2

        lax.fori_loop(0, G // L, group, 0, unroll=2)

    def _maybe(cond, fn):
        if isinstance(cond, bool):
            if cond:
                fn()
        else:
            pl.when(cond)(fn)

    def step(k, S, Snxt, do_prefetch, do_idx, not_first):
        def _pf():
            drain_idx(Snxt)
            issue_gather(Snxt)
        _maybe(do_prefetch, _pf)
        drain_gather(S)

        def _ds():
            pltpu.make_async_copy(S["ex"], ex_hbm.at[pl.ds(0, G)],
                                  S["semS"]).wait()
        _maybe(not_first, _ds)
        compute(k, S)
        pltpu.sync_copy(S["ex"], den_sp.at[S["dst"]], add=True)
        base_e = (wid * KT + k) * G
        pltpu.async_copy(S["ex"], ex_hbm.at[pl.ds(base_e, G)], S["semS"])

        def _ni():
            issue_idx(k + 2, S)
        _maybe(do_idx, _ni)

    issue_idx(0, SL[0])
    drain_idx(SL[0])
    issue_gather(SL[0])
    issue_idx(1, SL[1])

    def pair(i, carry):
        nl = i < KT // 2 - 1
        step(2 * i, SL[0], SL[1], True, nl, i > 0)
        step(2 * i + 1, SL[1], SL[0], nl, nl, i > 0)
        return carry

    lax.fori_loop(0, KT // 2, pair, 0)
    for S in SL:
        pltpu.make_async_copy(S["ex"], ex_hbm.at[pl.ds(0, G)], S["semS"]).wait()
    plsc.subcore_barrier()

    @pl.when(s == 0)
    def _fin():
        pltpu.sync_copy(den_sp, dpart_hbm.at[c])


def _rd_body(dp_ref, rd_ref):
    r = 1.0 / (dp_ref[0] + dp_ref[1])
    rd_ref[...] = jnp.concatenate([r, r], axis=-1)


def _stage_c(dparts):
    blk = 400
    return pl.pallas_call(
        _rd_body,
        grid=(N // blk,),
        in_specs=[pl.BlockSpec((NC, blk, H), lambda i: (0, i, 0))],
        out_specs=pl.BlockSpec((blk, 2 * H), lambda i: (i, 0)),
        out_shape=jax.ShapeDtypeStruct((N, 2 * H), jnp.float32),
    )(dparts)


def _bcast(v, j):
    idx = jnp.full((L, 1), j, jnp.int32)
    dn = lax.GatherDimensionNumbers(offset_dims=(), collapsed_slice_dims=(0,),
                                    start_index_map=(0,))
    return lax.gather(v, idx, dn, (1,),
                      mode=lax.GatherScatterMode.PROMISE_IN_BOUNDS)


@functools.partial(
    pl.kernel,
    out_type=jax.ShapeDtypeStruct((NC, N, OUT), jnp.float32),
    mesh=plsc.VectorSubcoreMesh(**_MESH),
    compiler_params=_SC_PARAMS,
    scratch_types=[
        pltpu.VMEM((G,), jnp.int32), pltpu.VMEM((G,), jnp.int32),
        pltpu.VMEM((G,), jnp.int32), pltpu.VMEM((G,), jnp.int32),
        pltpu.VMEM((G, OUT), jnp.float32), pltpu.VMEM((G, OUT), jnp.float32),
        pltpu.VMEM((G, H), jnp.float32), pltpu.VMEM((G, H), jnp.float32),
        pltpu.VMEM((G, 2 * H), jnp.float32), pltpu.VMEM((G, 2 * H), jnp.float32),
        pltpu.VMEM_SHARED((N, OUT), jnp.float32),
        pltpu.SemaphoreType.DMA, pltpu.SemaphoreType.DMA,
        pltpu.SemaphoreType.DMA, pltpu.SemaphoreType.DMA,
        pltpu.SemaphoreType.DMA, pltpu.SemaphoreType.DMA,
    ],
)
def _pass2(xh_hbm, srcp_hbm, dstp_hbm, ex_hbm, rd_hbm, zeroO_hbm, opart_hbm,
           srcA, srcB, dstA, dstB, rowsA, rowsB, exA, exB, rdA, rdB, out_sp,
           semIA, semIB, semGA, semGB, semDA, semDB):
    c = lax.axis_index("c")
    s = lax.axis_index("s")
    wid = c * NS + s

    @pl.when(s == 0)
    def _init():
        pltpu.sync_copy(zeroO_hbm, out_sp)

    plsc.subcore_barrier()
    iota = lax.iota(jnp.int32, L)
    mask8 = iota < H

    SL = (dict(src=srcA, dst=dstA, rows=rowsA, ex=exA, rd=rdA,
               semI=semIA, semG=semGA, semD=semDA),
          dict(src=srcB, dst=dstB, rows=rowsB, ex=exB, rd=rdB,
               semI=semIB, semG=semGB, semD=semDB))

    def issue_idx(k, S):
        kr = wid * KT + k
        pltpu.async_copy(srcp_hbm.at[kr], S["src"], S["semI"])
        pltpu.async_copy(dstp_hbm.at[kr], S["dst"], S["semI"])
        pltpu.async_copy(ex_hbm.at[pl.ds(kr * G, G)], S["ex"], S["semI"])

    def drain_idx(S):
        pltpu.make_async_copy(srcp_hbm.at[0], S["src"], S["semI"]).wait()
        pltpu.make_async_copy(dstp_hbm.at[0], S["dst"], S["semI"]).wait()
        pltpu.make_async_copy(ex_hbm.at[pl.ds(0, G)], S["ex"], S["semI"]).wait()

    def issue_gather(S):
        pltpu.async_copy(xh_hbm.at[S["src"]], S["rows"], S["semG"])
        pltpu.async_copy(rd_hbm.at[S["dst"]], S["rd"], S["semD"])

    def drain_gather(S):
        pltpu.make_async_copy(xh_hbm.at[pl.ds(0, G)], S["rows"], S["semG"]).wait()
        pltpu.make_async_copy(rd_hbm.at[pl.ds(0, G)], S["rd"], S["semD"]).wait()

    def compute(S):
        rows_v, ex_v, rd_v = S["rows"], S["ex"], S["rd"]

        def edge(e, carry2):
            ef = jnp.full((L,), e, jnp.int32)
            ex_row = plsc.load_gather(ex_v, [ef, iota], mask=mask8)
            rd_row = plsc.load_gather(rd_v, [ef, iota])
            att = ex_row * rd_row
            for j in range(H):
                bj = _bcast(att, j)
                r = rows_v[e, pl.ds(j * L, L)]
                rows_v[e, pl.ds(j * L, L)] = r * bj
            return carry2

        lax.fori_loop(0, G, edge, 0, unroll=4)

    def _maybe(cond, fn):
        if isinstance(cond, bool):
            if cond:
                fn()
        else:
            pl.when(cond)(fn)

    def step(k, S, Snxt, do_prefetch, do_idx):
        def _pf():
            drain_idx(Snxt)
            issue_gather(Snxt)
        _maybe(do_prefetch, _pf)
        drain_gather(S)
        compute(S)
        pltpu.sync_copy(S["rows"], out_sp.at[S["dst"]], add=True)

        def _ni():
            issue_idx(k + 2, S)
        _maybe(do_idx, _ni)

    issue_idx(0, SL[0])
    drain_idx(SL[0])
    issue_gather(SL[0])
    issue_idx(1, SL[1])

    def pair(i, carry):
        nl = i < KT // 2 - 1
        step(2 * i, SL[0], SL[1], True, nl)
        step(2 * i + 1, SL[1], SL[0], nl, nl)
        return carry

    lax.fori_loop(0, KT // 2, pair, 0)
    plsc.subcore_barrier()

    @pl.when(s == 0)
    def _fin():
        pltpu.sync_copy(out_sp, opart_hbm.at[c])


def _fin_body(op_ref, x_ref, b_ref, g_ref, be_ref, o_ref):
    y = op_ref[0] + op_ref[1] + b_ref[...] + x_ref[...]
    mean = jnp.mean(y, axis=-1, keepdims=True)
    var = jnp.mean((y - mean) ** 2, axis=-1, keepdims=True)
    o_ref[...] = (y - mean) * lax.rsqrt(var + 1e-5) * g_ref[...] + be_ref[...]


def _stage_e(oparts, x, bias, gamma, beta):
    blk = 400
    return pl.pallas_call(
        _fin_body,
        grid=(N // blk,),
        in_specs=[
            pl.BlockSpec((NC, blk, OUT), lambda i: (0, i, 0)),
            pl.BlockSpec((blk, OUT), lambda i: (i, 0)),
            pl.BlockSpec((1, OUT), lambda i: (0, 0)),
            pl.BlockSpec((1, OUT), lambda i: (0, 0)),
            pl.BlockSpec((1, OUT), lambda i: (0, 0)),
        ],
        out_specs=pl.BlockSpec((blk, OUT), lambda i: (i, 0)),
        out_shape=jax.ShapeDtypeStruct((N, OUT), jnp.float32),
    )(oparts, x, bias, gamma, beta)


def kernel(x, edge_index, W, att_src, att_dst, bias, gamma, beta):
    src = edge_index[0]
    dst = edge_index[1]
    loop = jnp.arange(N, dtype=jnp.int32)
    pad = jnp.arange(EP - NE, dtype=jnp.int32) % N
    srcp = jnp.concatenate([src, loop, pad]).reshape(ER, G)
    dstp = jnp.concatenate([dst, loop, pad]).reshape(ER, G)

    eyeH = jnp.eye(H, dtype=jnp.float32)
    A1 = (att_src[:, :, None] * eyeH[:, None, :]).reshape(OUT, H)
    A2 = (att_dst[:, :, None] * eyeH[:, None, :]).reshape(OUT, H)
    A = jnp.concatenate([A1, A2], axis=1)

    xh, ad = _stage_a(x, W, A)
    zero8 = jnp.zeros((N, H), jnp.float32)
    ex, dparts = _pass1(ad, srcp, dstp, zero8)
    rd = _stage_c(dparts)
    zeroO = jnp.zeros((N, OUT), jnp.float32)
    oparts = _pass2(xh, srcp, dstp, ex, rd, zeroO)
    return _stage_e(oparts, x, bias.reshape(1, OUT), gamma.reshape(1, OUT),
                    beta.reshape(1, OUT))


# speedup vs baseline: 1.3734x; 1.3734x over previous
import functools

import jax
import jax.numpy as jnp
from jax import lax
from jax.experimental import pallas as pl
from jax.experimental.pallas import tpu as pltpu
from jax.experimental.pallas import tpu_sc as plsc

N = 10000
E = 320000
IN = 128
OUT = 128
H = 8
C = OUT // H
NE = E + N
NC, NS, L = 2, 16, 16
NW = NC * NS
G = 128
KT = 84
EP = NW * G * KT
ER = EP // G
NEG = 0.2

_MESH = dict(core_axis_name="c", subcore_axis_name="s", num_cores=NC,
             num_subcores=NS)
_SC_PARAMS = pltpu.CompilerParams(needs_layout_passes=False,
                                  use_tc_tiling_on_sc=False)


def _mm_body(x_ref, w_ref, a_ref, xh_ref, ad_ref):
    xh = jnp.dot(x_ref[...], w_ref[...], preferred_element_type=jnp.float32)
    xh_ref[...] = xh
    ad_ref[...] = jnp.dot(xh, a_ref[...], preferred_element_type=jnp.float32)


def _stage_a(x, W, A):
    blk = 400
    return pl.pallas_call(
        _mm_body,
        grid=(N // blk,),
        in_specs=[
            pl.BlockSpec((blk, IN), lambda i: (i, 0)),
            pl.BlockSpec((IN, OUT), lambda i: (0, 0)),
            pl.BlockSpec((OUT, 2 * H), lambda i: (0, 0)),
        ],
        out_specs=[
            pl.BlockSpec((blk, OUT), lambda i: (i, 0)),
            pl.BlockSpec((blk, 2 * H), lambda i: (i, 0)),
        ],
        out_shape=[
            jax.ShapeDtypeStruct((N, OUT), jnp.float32),
            jax.ShapeDtypeStruct((N, 2 * H), jnp.float32),
        ],
    )(x, W, A)


@functools.partial(
    pl.kernel,
    out_type=(jax.ShapeDtypeStruct((EP, H), jnp.float32),
              jax.ShapeDtypeStruct((N, H), jnp.float32),
              jax.ShapeDtypeStruct((N, H), jnp.float32)),
    mesh=plsc.VectorSubcoreMesh(**_MESH),
    compiler_params=_SC_PARAMS,
    scratch_types=[
        pltpu.VMEM((G,), jnp.int32), pltpu.VMEM((G,), jnp.int32),
        pltpu.VMEM((G,), jnp.int32), pltpu.VMEM((G,), jnp.int32),
        pltpu.VMEM((G, 2 * H), jnp.float32), pltpu.VMEM((G, 2 * H), jnp.float32),
        pltpu.VMEM((G, 2 * H), jnp.float32), pltpu.VMEM((G, 2 * H), jnp.float32),
        pltpu.VMEM((G, H), jnp.float32), pltpu.VMEM((G, H), jnp.float32),
        pltpu.VMEM_SHARED((N, H), jnp.float32),
        pltpu.SemaphoreType.DMA, pltpu.SemaphoreType.DMA,
        pltpu.SemaphoreType.DMA, pltpu.SemaphoreType.DMA,
        pltpu.SemaphoreType.DMA, pltpu.SemaphoreType.DMA,
    ],
)
def _pass1(ad_hbm, srcp_hbm, dstp_hbm, zero8_hbm, ex_hbm, d0_hbm, d1_hbm,
           srcA, srcB, dstA, dstB, adsA, adsB, addA, addB, exA, exB, den_sp,
           semIA, semIB, semGA, semGB, semSA, semSB):
    c = lax.axis_index("c")
    s = lax.axis_index("s")
    wid = c * NS + s

    @pl.when(s == 0)
    def _init():
        pltpu.sync_copy(zero8_hbm, den_sp)

    plsc.subcore_barrier()
    iota = lax.iota(jnp.int32, L)

    SL = (dict(src=srcA, dst=dstA, ads=adsA, add=addA, ex=exA,
               semI=semIA, semG=semGA, semS=semSA),
          dict(src=srcB, dst=dstB, ads=adsB, add=addB, ex=exB,
               semI=semIB, semG=semGB, semS=semSB))

    def issue_idx(k, S):
        kr = wid * KT + k
        pltpu.async_copy(srcp_hbm.at[kr], S["src"], S["semI"])
        pltpu.async_copy(dstp_hbm.at[kr], S["dst"], S["semI"])

    def drain_idx(S):
        pltpu.make_async_copy(srcp_hbm.at[0], S["src"], S["semI"]).wait()
        pltpu.make_async_copy(dstp_hbm.at[0], S["dst"], S["semI"]).wait()

    def issue_gather(S):
        pltpu.async_copy(ad_hbm.at[S["src"]], S["ads"], S["semG"])
        pltpu.async_copy(ad_hbm.at[S["dst"]], S["add"], S["semG"])

    def drain_gather(S):
        pltpu.make_async_copy(ad_hbm.at[pl.ds(0, G)], S["ads"], S["semG"]).wait()
        pltpu.make_async_copy(ad_hbm.at[pl.ds(0, G)], S["add"], S["semG"]).wait()

    def compute(k, S):
        base_e = (wid * KT + k) * G
        ads_v, add_v, ex_v = S["ads"], S["add"], S["ex"]

        def group(g, carry2):
            e_loc = g * L + iota
            mask = (base_e + e_loc) < NE
            for h in range(H):
                hs = jnp.full((L,), h, jnp.int32)
                hd = jnp.full((L,), H + h, jnp.int32)
                a = (plsc.load_gather(ads_v, [e_loc, hs])
                     + plsc.load_gather(add_v, [e_loc, hd]))
                a = jnp.where(a < 0, a * NEG, a)
                exv = jnp.where(mask, jnp.exp(a), 0.0)
                plsc.store_scatter(ex_v, [e_loc, hs], exv)
            return carry2

        lax.fori_loop(0, G // L, group, 0)

    def _maybe(cond, fn):
        if isinstance(cond, bool):
            if cond:
                fn()
        else:
            pl.when(cond)(fn)

    def step(k, S, Snxt, do_prefetch, do_idx, not_first):
        def _pf():
            drain_idx(Snxt)
            issue_gather(Snxt)
        _maybe(do_prefetch, _pf)
        drain_gather(S)

        def _ds():
            pltpu.make_async_copy(S["ex"], ex_hbm.at[pl.ds(0, G)],
                                  S["semS"]).wait()
        _maybe(not_first, _ds)
        compute(k, S)
        pltpu.sync_copy(S["ex"], den_sp.at[S["dst"]], add=True)
        base_e = (wid * KT + k) * G
        pltpu.async_copy(S["ex"], ex_hbm.at[pl.ds(base_e, G)], S["semS"])

        def _ni():
            issue_idx(k + 2, S)
        _maybe(do_idx, _ni)

    issue_idx(0, SL[0])
    drain_idx(SL[0])
    issue_gather(SL[0])
    issue_idx(1, SL[1])

    def pair(i, carry):
        nl = i < KT // 2 - 1
        step(2 * i, SL[0], SL[1], True, nl, i > 0)
        step(2 * i + 1, SL[1], SL[0], nl, nl, i > 0)
        return carry

    lax.fori_loop(0, KT // 2, pair, 0)
    for S in SL:
        pltpu.make_async_copy(S["ex"], ex_hbm.at[pl.ds(0, G)], S["semS"]).wait()
    plsc.subcore_barrier()

    @pl.when(jnp.logical_and(s == 0, c == 0))
    def _fin0():
        pltpu.sync_copy(den_sp, d0_hbm)

    @pl.when(jnp.logical_and(s == 0, c == 1))
    def _fin1():
        pltpu.sync_copy(den_sp, d1_hbm)


def _bcast(v, j):
    idx = jnp.full((L, 1), j, jnp.int32)
    dn = lax.GatherDimensionNumbers(offset_dims=(), collapsed_slice_dims=(0,),
                                    start_index_map=(0,))
    return lax.gather(v, idx, dn, (1,),
                      mode=lax.GatherScatterMode.PROMISE_IN_BOUNDS)


@functools.partial(
    pl.kernel,
    out_type=jax.ShapeDtypeStruct((NC, N, OUT), jnp.float32),
    mesh=plsc.VectorSubcoreMesh(**_MESH),
    compiler_params=_SC_PARAMS,
    scratch_types=[
        pltpu.VMEM((G,), jnp.int32), pltpu.VMEM((G,), jnp.int32),
        pltpu.VMEM((G,), jnp.int32), pltpu.VMEM((G,), jnp.int32),
        pltpu.VMEM((G, OUT), jnp.float32), pltpu.VMEM((G, OUT), jnp.float32),
        pltpu.VMEM((G, H), jnp.float32), pltpu.VMEM((G, H), jnp.float32),
        pltpu.VMEM((G, H), jnp.float32), pltpu.VMEM((G, H), jnp.float32),
        pltpu.VMEM((G, H), jnp.float32), pltpu.VMEM((G, H), jnp.float32),
        pltpu.VMEM_SHARED((N, OUT), jnp.float32),
        pltpu.SemaphoreType.DMA, pltpu.SemaphoreType.DMA,
        pltpu.SemaphoreType.DMA, pltpu.SemaphoreType.DMA,
        pltpu.SemaphoreType.DMA, pltpu.SemaphoreType.DMA,
    ],
)
def _pass2(xh_hbm, srcp_hbm, dstp_hbm, ex_hbm, d0_hbm, d1_hbm, zeroO_hbm,
           opart_hbm,
           srcA, srcB, dstA, dstB, rowsA, rowsB, exA, exB,
           d0A, d0B, d1A, d1B, out_sp,
           semIA, semIB, semGA, semGB, semDA, semDB):
    c = lax.axis_index("c")
    s = lax.axis_index("s")
    wid = c * NS + s

    @pl.when(s == 0)
    def _init():
        pltpu.sync_copy(zeroO_hbm, out_sp)

    plsc.subcore_barrier()
    iota = lax.iota(jnp.int32, L)
    mask8 = iota < H

    SL = (dict(src=srcA, dst=dstA, rows=rowsA, ex=exA, d0=d0A, d1=d1A,
               semI=semIA, semG=semGA, semD=semDA),
          dict(src=srcB, dst=dstB, rows=rowsB, ex=exB, d0=d0B, d1=d1B,
               semI=semIB, semG=semGB, semD=semDB))

    def issue_idx(k, S):
        kr = wid * KT + k
        pltpu.async_copy(srcp_hbm.at[kr], S["src"], S["semI"])
        pltpu.async_copy(dstp_hbm.at[kr], S["dst"], S["semI"])
        pltpu.async_copy(ex_hbm.at[pl.ds(kr * G, G)], S["ex"], S["semI"])

    def drain_idx(S):
        pltpu.make_async_copy(srcp_hbm.at[0], S["src"], S["semI"]).wait()
        pltpu.make_async_copy(dstp_hbm.at[0], S["dst"], S["semI"]).wait()
        pltpu.make_async_copy(ex_hbm.at[pl.ds(0, G)], S["ex"], S["semI"]).wait()

    def issue_gather(S):
        pltpu.async_copy(xh_hbm.at[S["src"]], S["rows"], S["semG"])
        pltpu.async_copy(d0_hbm.at[S["dst"]], S["d0"], S["semD"])
        pltpu.async_copy(d1_hbm.at[S["dst"]], S["d1"], S["semD"])

    def drain_gather(S):
        pltpu.make_async_copy(xh_hbm.at[pl.ds(0, G)], S["rows"], S["semG"]).wait()
        pltpu.make_async_copy(d0_hbm.at[pl.ds(0, G)], S["d0"], S["semD"]).wait()
        pltpu.make_async_copy(d1_hbm.at[pl.ds(0, G)], S["d1"], S["semD"]).wait()

    def compute(S):
        rows_v, ex_v, d0_v, d1_v = S["rows"], S["ex"], S["d0"], S["d1"]

        def edge(e, carry2):
            ef = jnp.full((L,), e, jnp.int32)
            ex_row = plsc.load_gather(ex_v, [ef, iota], mask=mask8)
            den = (plsc.load_gather(d0_v, [ef, iota], mask=mask8)
                   + plsc.load_gather(d1_v, [ef, iota], mask=mask8))
            att = ex_row / den
            for j in range(H):
                bj = _bcast(att, j)
                r = rows_v[e, pl.ds(j * L, L)]
                rows_v[e, pl.ds(j * L, L)] = r * bj
            return carry2

        lax.fori_loop(0, G, edge, 0)

    def _maybe(cond, fn):
        if isinstance(cond, bool):
            if cond:
                fn()
        else:
            pl.when(cond)(fn)

    def step(k, S, Snxt, do_prefetch, do_idx):
        def _pf():
            drain_idx(Snxt)
            issue_gather(Snxt)
        _maybe(do_prefetch, _pf)
        drain_gather(S)
        compute(S)
        pltpu.sync_copy(S["rows"], out_sp.at[S["dst"]], add=True)

        def _ni():
            issue_idx(k + 2, S)
        _maybe(do_idx, _ni)

    issue_idx(0, SL[0])
    drain_idx(SL[0])
    issue_gather(SL[0])
    issue_idx(1, SL[1])

    def pair(i, carry):
        nl = i < KT // 2 - 1
        step(2 * i, SL[0], SL[1], True, nl)
        step(2 * i + 1, SL[1], SL[0], nl, nl)
        return carry

    lax.fori_loop(0, KT // 2, pair, 0)
    plsc.subcore_barrier()

    @pl.when(s == 0)
    def _fin():
        pltpu.sync_copy(out_sp, opart_hbm.at[c])


def _fin_body(op_ref, x_ref, b_ref, g_ref, be_ref, o_ref):
    y = op_ref[0] + op_ref[1] + b_ref[...] + x_ref[...]
    mean = jnp.mean(y, axis=-1, keepdims=True)
    var = jnp.mean((y - mean) ** 2, axis=-1, keepdims=True)
    o_ref[...] = (y - mean) * lax.rsqrt(var + 1e-5) * g_ref[...] + be_ref[...]


def _stage_e(oparts, x, bias, gamma, beta):
    blk = 400
    return pl.pallas_call(
        _fin_body,
        grid=(N // blk,),
        in_specs=[
            pl.BlockSpec((NC, blk, OUT), lambda i: (0, i, 0)),
            pl.BlockSpec((blk, OUT), lambda i: (i, 0)),
            pl.BlockSpec((1, OUT), lambda i: (0, 0)),
            pl.BlockSpec((1, OUT), lambda i: (0, 0)),
            pl.BlockSpec((1, OUT), lambda i: (0, 0)),
        ],
        out_specs=pl.BlockSpec((blk, OUT), lambda i: (i, 0)),
        out_shape=jax.ShapeDtypeStruct((N, OUT), jnp.float32),
    )(oparts, x, bias, gamma, beta)


def kernel(x, edge_index, W, att_src, att_dst, bias, gamma, beta):
    src = edge_index[0]
    dst = edge_index[1]
    loop = jnp.arange(N, dtype=jnp.int32)
    pad = jnp.arange(EP - NE, dtype=jnp.int32) % N
    srcp = jnp.concatenate([src, loop, pad]).reshape(ER, G)
    dstp = jnp.concatenate([dst, loop, pad]).reshape(ER, G)

    eyeH = jnp.eye(H, dtype=jnp.float32)
    A1 = (att_src[:, :, None] * eyeH[:, None, :]).reshape(OUT, H)
    A2 = (att_dst[:, :, None] * eyeH[:, None, :]).reshape(OUT, H)
    A = jnp.concatenate([A1, A2], axis=1)

    xh, ad = _stage_a(x, W, A)
    zero8 = jnp.zeros((N, H), jnp.float32)
    ex, d0, d1 = _pass1(ad, srcp, dstp, zero8)
    zeroO = jnp.zeros((N, OUT), jnp.float32)
    oparts = _pass2(xh, srcp, dstp, ex, d0, d1, zeroO)
    return _stage_e(oparts, x, bias.reshape(1, OUT), gamma.reshape(1, OUT),
                    beta.reshape(1, OUT))


# speedup vs baseline: 1.5137x; 1.1022x over previous
import functools

import jax
import jax.numpy as jnp
from jax import lax
from jax.experimental import pallas as pl
from jax.experimental.pallas import tpu as pltpu
from jax.experimental.pallas import tpu_sc as plsc

N = 10000
E = 320000
IN = 128
OUT = 128
H = 8
C = OUT // H
NE = E + N
NC, NS, L = 2, 16, 16
NW = NC * NS
G = 128
KT = 82
EP = NW * G * KT
ER = EP // G
NEG = 0.2

_MESH = dict(core_axis_name="c", subcore_axis_name="s", num_cores=NC,
             num_subcores=NS)
_SC_PARAMS = pltpu.CompilerParams(needs_layout_passes=False,
                                  use_tc_tiling_on_sc=False)


def _mm_body(x_ref, w_ref, a_ref, xh_ref, ad_ref):
    xh = jnp.dot(x_ref[...], w_ref[...], preferred_element_type=jnp.float32)
    xh_ref[...] = xh
    ad_ref[...] = jnp.dot(xh, a_ref[...], preferred_element_type=jnp.float32)


def _stage_a(x, W, A):
    blk = 400
    return pl.pallas_call(
        _mm_body,
        grid=(N // blk,),
        in_specs=[
            pl.BlockSpec((blk, IN), lambda i: (i, 0)),
            pl.BlockSpec((IN, OUT), lambda i: (0, 0)),
            pl.BlockSpec((OUT, 2 * H), lambda i: (0, 0)),
        ],
        out_specs=[
            pl.BlockSpec((blk, OUT), lambda i: (i, 0)),
            pl.BlockSpec((blk, 2 * H), lambda i: (i, 0)),
        ],
        out_shape=[
            jax.ShapeDtypeStruct((N, OUT), jnp.float32),
            jax.ShapeDtypeStruct((N, 2 * H), jnp.float32),
        ],
    )(x, W, A)


@functools.partial(
    pl.kernel,
    out_type=(jax.ShapeDtypeStruct((EP, H), jnp.float32),
              jax.ShapeDtypeStruct((NC, N, H), jnp.float32)),
    mesh=plsc.VectorSubcoreMesh(**_MESH),
    compiler_params=_SC_PARAMS,
    scratch_types=[
        pltpu.VMEM((G,), jnp.int32), pltpu.VMEM((G,), jnp.int32),
        pltpu.VMEM((G,), jnp.int32), pltpu.VMEM((G,), jnp.int32),
        pltpu.VMEM((G, 2 * H), jnp.float32), pltpu.VMEM((G, 2 * H), jnp.float32),
        pltpu.VMEM((G, 2 * H), jnp.float32), pltpu.VMEM((G, 2 * H), jnp.float32),
        pltpu.VMEM((G, H), jnp.float32), pltpu.VMEM((G, H), jnp.float32),
        pltpu.VMEM_SHARED((N, H), jnp.float32),
        pltpu.SemaphoreType.DMA, pltpu.SemaphoreType.DMA,
        pltpu.SemaphoreType.DMA, pltpu.SemaphoreType.DMA,
        pltpu.SemaphoreType.DMA, pltpu.SemaphoreType.DMA,
    ],
)
def _pass1(ad_hbm, srcp_hbm, dstp_hbm, zero8_hbm, ex_hbm, dpart_hbm,
           srcA, srcB, dstA, dstB, adsA, adsB, addA, addB, exA, exB, den_sp,
           semIA, semIB, semGA, semGB, semSA, semSB):
    c = lax.axis_index("c")
    s = lax.axis_index("s")
    wid = c * NS + s

    @pl.when(s == 0)
    def _init():
        pltpu.sync_copy(zero8_hbm, den_sp)

    plsc.subcore_barrier()
    iota = lax.iota(jnp.int32, L)

    SL = (dict(src=srcA, dst=dstA, ads=adsA, add=addA, ex=exA,
               semI=semIA, semG=semGA, semS=semSA),
          dict(src=srcB, dst=dstB, ads=adsB, add=addB, ex=exB,
               semI=semIB, semG=semGB, semS=semSB))

    def issue_idx(k, S):
        kr = wid * KT + k
        pltpu.async_copy(srcp_hbm.at[kr], S["src"], S["semI"])
        pltpu.async_copy(dstp_hbm.at[kr], S["dst"], S["semI"])

    def drain_idx(S):
        pltpu.make_async_copy(srcp_hbm.at[0], S["src"], S["semI"]).wait()
        pltpu.make_async_copy(dstp_hbm.at[0], S["dst"], S["semI"]).wait()

    def issue_gather(S):
        pltpu.async_copy(ad_hbm.at[S["src"]], S["ads"], S["semG"])
        pltpu.async_copy(ad_hbm.at[S["dst"]], S["add"], S["semG"])

    def drain_gather(S):
        pltpu.make_async_copy(ad_hbm.at[pl.ds(0, G)], S["ads"], S["semG"]).wait()
        pltpu.make_async_copy(ad_hbm.at[pl.ds(0, G)], S["add"], S["semG"]).wait()

    def compute(k, S):
        base_e = (wid * KT + k) * G
        ads_v, add_v, ex_v = S["ads"], S["add"], S["ex"]

        def group(g, carry2):
            e_loc = g * L + iota
            mask = (base_e + e_loc) < NE
            for h in range(H):
                hs = jnp.full((L,), h, jnp.int32)
                hd = jnp.full((L,), H + h, jnp.int32)
                a = (plsc.load_gather(ads_v, [e_loc, hs])
                     + plsc.load_gather(add_v, [e_loc, hd]))
                a = jnp.where(a < 0, a * NEG, a)
                exv = jnp.where(mask, jnp.exp(a), 0.0)
                plsc.store_scatter(ex_v, [e_loc, hs], exv)
            return carry2

        lax.fori_loop(0, G // L, group, 0)

    def _maybe(cond, fn):
        if isinstance(cond, bool):
            if cond:
                fn()
        else:
            pl.when(cond)(fn)

    def step(k, S, Snxt, do_prefetch, do_idx, not_first):
        def _pf():
            drain_idx(Snxt)
            issue_gather(Snxt)
        _maybe(do_prefetch, _pf)
        drain_gather(S)

        def _ds():
            pltpu.make_async_copy(S["ex"], ex_hbm.at[pl.ds(0, G)],
                                  S["semS"]).wait()
        _maybe(not_first, _ds)
        compute(k, S)
        pltpu.sync_copy(S["ex"], den_sp.at[S["dst"]], add=True)
        base_e = (wid * KT + k) * G
        pltpu.async_copy(S["ex"], ex_hbm.at[pl.ds(base_e, G)], S["semS"])

        def _ni():
            issue_idx(k + 2, S)
        _maybe(do_idx, _ni)

    issue_idx(0, SL[0])
    drain_idx(SL[0])
    issue_gather(SL[0])
    issue_idx(1, SL[1])

    def pair(i, carry):
        nl = i < KT // 2 - 1
        step(2 * i, SL[0], SL[1], True, nl, i > 0)
        step(2 * i + 1, SL[1], SL[0], nl, nl, i > 0)
        return carry

    lax.fori_loop(0, KT // 2, pair, 0)
    for S in SL:
        pltpu.make_async_copy(S["ex"], ex_hbm.at[pl.ds(0, G)], S["semS"]).wait()
    plsc.subcore_barrier()

    @pl.when(s == 0)
    def _fin():
        pltpu.sync_copy(den_sp, dpart_hbm.at[c])


def _rd_body(dp_ref, rd_ref):
    r = 1.0 / (dp_ref[0] + dp_ref[1])
    rd_ref[...] = jnp.concatenate([r, r], axis=-1)


def _stage_c(dparts):
    blk = 400
    return pl.pallas_call(
        _rd_body,
        grid=(N // blk,),
        in_specs=[pl.BlockSpec((NC, blk, H), lambda i: (0, i, 0))],
        out_specs=pl.BlockSpec((blk, 2 * H), lambda i: (i, 0)),
        out_shape=jax.ShapeDtypeStruct((N, 2 * H), jnp.float32),
    )(dparts)


def _bcast(v, j):
    idx = jnp.full((L, 1), j, jnp.int32)
    dn = lax.GatherDimensionNumbers(offset_dims=(), collapsed_slice_dims=(0,),
                                    start_index_map=(0,))
    return lax.gather(v, idx, dn, (1,),
                      mode=lax.GatherScatterMode.PROMISE_IN_BOUNDS)


@functools.partial(
    pl.kernel,
    out_type=jax.ShapeDtypeStruct((NC, N, OUT), jnp.float32),
    mesh=plsc.VectorSubcoreMesh(**_MESH),
    compiler_params=_SC_PARAMS,
    scratch_types=[
        pltpu.VMEM((G,), jnp.int32), pltpu.VMEM((G,), jnp.int32),
        pltpu.VMEM((G,), jnp.int32), pltpu.VMEM((G,), jnp.int32),
        pltpu.VMEM((G, OUT), jnp.float32), pltpu.VMEM((G, OUT), jnp.float32),
        pltpu.VMEM((G, H), jnp.float32), pltpu.VMEM((G, H), jnp.float32),
        pltpu.VMEM((G, 2 * H), jnp.float32), pltpu.VMEM((G, 2 * H), jnp.float32),
        pltpu.VMEM_SHARED((N, OUT), jnp.float32),
        pltpu.SemaphoreType.DMA, pltpu.SemaphoreType.DMA,
        pltpu.SemaphoreType.DMA, pltpu.SemaphoreType.DMA,
        pltpu.SemaphoreType.DMA, pltpu.SemaphoreType.DMA,
    ],
)
def _pass2(xh_hbm, srcp_hbm, dstp_hbm, ex_hbm, rd_hbm, zeroO_hbm, opart_hbm,
           srcA, srcB, dstA, dstB, rowsA, rowsB, exA, exB, rdA, rdB, out_sp,
           semIA, semIB, semGA, semGB, semDA, semDB):
    c = lax.axis_index("c")
    s = lax.axis_index("s")
    wid = c * NS + s

    @pl.when(s == 0)
    def _init():
        pltpu.sync_copy(zeroO_hbm, out_sp)

    plsc.subcore_barrier()
    iota = lax.iota(jnp.int32, L)
    mask8 = iota < H

    SL = (dict(src=srcA, dst=dstA, rows=rowsA, ex=exA, rd=rdA,
               semI=semIA, semG=semGA, semD=semDA),
          dict(src=srcB, dst=dstB, rows=rowsB, ex=exB, rd=rdB,
               semI=semIB, semG=semGB, semD=semDB))

    def issue_idx(k, S):
        kr = wid * KT + k
        pltpu.async_copy(srcp_hbm.at[kr], S["src"], S["semI"])
        pltpu.async_copy(dstp_hbm.at[kr], S["dst"], S["semI"])
        pltpu.async_copy(ex_hbm.at[pl.ds(kr * G, G)], S["ex"], S["semI"])

    def drain_idx(S):
        pltpu.make_async_copy(srcp_hbm.at[0], S["src"], S["semI"]).wait()
        pltpu.make_async_copy(dstp_hbm.at[0], S["dst"], S["semI"]).wait()
        pltpu.make_async_copy(ex_hbm.at[pl.ds(0, G)], S["ex"], S["semI"]).wait()

    def issue_gather(S):
        pltpu.async_copy(xh_hbm.at[S["src"]], S["rows"], S["semG"])
        pltpu.async_copy(rd_hbm.at[S["dst"]], S["rd"], S["semD"])

    def drain_gather(S):
        pltpu.make_async_copy(xh_hbm.at[pl.ds(0, G)], S["rows"], S["semG"]).wait()
        pltpu.make_async_copy(rd_hbm.at[pl.ds(0, G)], S["rd"], S["semD"]).wait()

    def compute(S):
        rows_v, ex_v, rd_v = S["rows"], S["ex"], S["rd"]

        def edge(e, carry2):
            ef = jnp.full((L,), e, jnp.int32)
            ex_row = plsc.load_gather(ex_v, [ef, iota], mask=mask8)
            rd_row = plsc.load_gather(rd_v, [ef, iota])
            att = ex_row * rd_row
            for j in range(H):
                bj = _bcast(att, j)
                r = rows_v[e, pl.ds(j * L, L)]
                rows_v[e, pl.ds(j * L, L)] = r * bj
            return carry2

        lax.fori_loop(0, G, edge, 0)

    def _maybe(cond, fn):
        if isinstance(cond, bool):
            if cond:
                fn()
        else:
            pl.when(cond)(fn)

    def step(k, S, Snxt, do_prefetch, do_idx):
        def _pf():
            drain_idx(Snxt)
            issue_gather(Snxt)
        _maybe(do_prefetch, _pf)
        drain_gather(S)
        compute(S)
        pltpu.sync_copy(S["rows"], out_sp.at[S["dst"]], add=True)

        def _ni():
            issue_idx(k + 2, S)
        _maybe(do_idx, _ni)

    issue_idx(0, SL[0])
    drain_idx(SL[0])
    issue_gather(SL[0])
    issue_idx(1, SL[1])

    def pair(i, carry):
        nl = i < KT // 2 - 1
        step(2 * i, SL[0], SL[1], True, nl)
        step(2 * i + 1, SL[1], SL[0], nl, nl)
        return carry

    lax.fori_loop(0, KT // 2, pair, 0)
    plsc.subcore_barrier()

    @pl.when(s == 0)
    def _fin():
        pltpu.sync_copy(out_sp, opart_hbm.at[c])


def _fin_body(op_ref, x_ref, b_ref, g_ref, be_ref, o_ref):
    y = op_ref[0] + op_ref[1] + b_ref[...] + x_ref[...]
    mean = jnp.mean(y, axis=-1, keepdims=True)
    var = jnp.mean((y - mean) ** 2, axis=-1, keepdims=True)
    o_ref[...] = (y - mean) * lax.rsqrt(var + 1e-5) * g_ref[...] + be_ref[...]


def _stage_e(oparts, x, bias, gamma, beta):
    blk = 400
    return pl.pallas_call(
        _fin_body,
        grid=(N // blk,),
        in_specs=[
            pl.BlockSpec((NC, blk, OUT), lambda i: (0, i, 0)),
            pl.BlockSpec((blk, OUT), lambda i: (i, 0)),
            pl.BlockSpec((1, OUT), lambda i: (0, 0)),
            pl.BlockSpec((1, OUT), lambda i: (0, 0)),
            pl.BlockSpec((1, OUT), lambda i: (0, 0)),
        ],
        out_specs=pl.BlockSpec((blk, OUT), lambda i: (i, 0)),
        out_shape=jax.ShapeDtypeStruct((N, OUT), jnp.float32),
    )(oparts, x, bias, gamma, beta)


def kernel(x, edge_index, W, att_src, att_dst, bias, gamma, beta):
    src = edge_index[0]
    dst = edge_index[1]
    loop = jnp.arange(N, dtype=jnp.int32)
    pad = jnp.arange(EP - NE, dtype=jnp.int32) % N
    srcp = jnp.concatenate([src, loop, pad]).reshape(ER, G)
    dstp = jnp.concatenate([dst, loop, pad]).reshape(ER, G)

    eyeH = jnp.eye(H, dtype=jnp.float32)
    A1 = (att_src[:, :, None] * eyeH[:, None, :]).reshape(OUT, H)
    A2 = (att_dst[:, :, None] * eyeH[:, None, :]).reshape(OUT, H)
    A = jnp.concatenate([A1, A2], axis=1)

    xh, ad = _stage_a(x, W, A)
    zero8 = jnp.zeros((N, H), jnp.float32)
    ex, dparts = _pass1(ad, srcp, dstp, zero8)
    rd = _stage_c(dparts)
    zeroO = jnp.zeros((N, OUT), jnp.float32)
    oparts = _pass2(xh, srcp, dstp, ex, rd, zeroO)
    return _stage_e(oparts, x, bias.reshape(1, OUT), gamma.reshape(1, OUT),
                    beta.reshape(1, OUT))
